# baseline (speedup 1.0000x reference)
import jax
import jax.numpy as jnp
from jax.experimental import pallas as pl
from jax.experimental.pallas import tpu as pltpu


_TOPK = 10
_GX_CHUNKS = 4


def _policy_kernel(gx_hbm, slen_ref, whh_ref, wout_hbm, bout_ref,
                   probs_hbm, h_ref, c_ref, idx_ref,
                   gx_vmem, wout_vmem, probs_vmem, gx_sems, wout_sem,
                   probs_sem):
    T = gx_vmem.shape[0]
    Bb, H = h_ref.shape[1], h_ref.shape[2]
    C = wout_vmem.shape[1]
    tc = T // _GX_CHUNKS

    gx_cps = [
        pltpu.make_async_copy(gx_hbm.at[pl.ds(i * tc, tc)],
                              gx_vmem.at[pl.ds(i * tc, tc)],
                              gx_sems.at[i])
        for i in range(_GX_CHUNKS)
    ]
    for cp in gx_cps:
        cp.start()
    wout_cp = pltpu.make_async_copy(wout_hbm, wout_vmem, wout_sem)
    wout_cp.start()

    h0 = jnp.zeros((Bb, H), jnp.bfloat16)
    c0 = jnp.zeros((Bb, H), jnp.float32)
    slen = slen_ref[...]

    def step(t, carry):
        h, c = carry
        gates = gx_vmem[t].astype(jnp.float32) + jnp.dot(
            h, whh_ref[...], preferred_element_type=jnp.float32)
        i_g = jax.nn.sigmoid(gates[:, 0 * H:1 * H])
        f_g = jax.nn.sigmoid(gates[:, 1 * H:2 * H])
        g_g = jnp.tanh(gates[:, 2 * H:3 * H])
        o_g = jax.nn.sigmoid(gates[:, 3 * H:4 * H])
        c_new = f_g * c + i_g * g_g
        h_new = (o_g * jnp.tanh(c_new)).astype(jnp.bfloat16)
        valid = t < slen
        return (jnp.where(valid, h_new, h), jnp.where(valid, c_new, c))

    carry = (h0, c0)
    for i in range(_GX_CHUNKS):
        gx_cps[i].wait()
        carry = jax.lax.fori_loop(i * tc, (i + 1) * tc, step, carry,
                                  unroll=True)
    h, c = carry

    wout_cp.wait()
    hf = h.astype(jnp.float32)
    logits = jnp.dot(hf, wout_vmem[...],
                     preferred_element_type=jnp.float32) + bout_ref[...]
    m = jnp.max(logits, axis=1, keepdims=True)
    e = jnp.exp(logits - m)
    probs_vmem[...] = e * (1.0 / jnp.sum(e, axis=1, keepdims=True))
    probs_cp = pltpu.make_async_copy(probs_vmem, probs_hbm, probs_sem)
    probs_cp.start()
    h_ref[0] = hf
    c_ref[0] = c

    lane = jax.lax.broadcasted_iota(jnp.int32, (Bb, C), 1)
    vals = logits
    mk = m
    for k in range(_TOPK):
        idx_k = jnp.min(jnp.where(vals == mk, lane, C), axis=1, keepdims=True)
        idx_ref[:, k:k + 1] = idx_k
        if k + 1 < _TOPK:
            vals = jnp.where(lane == idx_k, -jnp.inf, vals)
            mk = jnp.max(vals, axis=1, keepdims=True)

    probs_cp.wait()


def kernel(seq_idx, seq_len, embedding, w_ih, w_hh, b, w_out, b_out):
    T, B = seq_idx.shape
    H = w_hh.shape[0]
    C = w_out.shape[1]
    Bb = B

    seq_em = jnp.take(embedding, seq_idx, axis=0).astype(jnp.float32)
    gates_x = (jnp.einsum("tbe,eg->tbg", seq_em, w_ih.astype(jnp.float32)) + b
               ).astype(jnp.bfloat16)
    slen = seq_len.astype(jnp.int32)[:, None]

    probs, h_last, c_last, indices = pl.pallas_call(
        _policy_kernel,
        out_shape=(
            jax.ShapeDtypeStruct((B, C), jnp.float32),
            jax.ShapeDtypeStruct((1, B, H), jnp.float32),
            jax.ShapeDtypeStruct((1, B, H), jnp.float32),
            jax.ShapeDtypeStruct((B, _TOPK), jnp.int32),
        ),
        grid=(1,),
        in_specs=[
            pl.BlockSpec(memory_space=pl.ANY),
            pl.BlockSpec((Bb, 1), lambda i: (i, 0)),
            pl.BlockSpec((H, 4 * H), lambda i: (0, 0)),
            pl.BlockSpec(memory_space=pl.ANY),
            pl.BlockSpec((1, C), lambda i: (0, 0)),
        ],
        out_specs=(
            pl.BlockSpec(memory_space=pl.ANY),
            pl.BlockSpec((1, Bb, H), lambda i: (0, i, 0)),
            pl.BlockSpec((1, Bb, H), lambda i: (0, i, 0)),
            pl.BlockSpec((Bb, _TOPK), lambda i: (i, 0)),
        ),
        scratch_shapes=[
            pltpu.VMEM((T, Bb, 4 * H), jnp.bfloat16),
            pltpu.VMEM((H, C), jnp.float32),
            pltpu.VMEM((Bb, C), jnp.float32),
            pltpu.SemaphoreType.DMA((_GX_CHUNKS,)),
            pltpu.SemaphoreType.DMA,
            pltpu.SemaphoreType.DMA,
        ],
        compiler_params=pltpu.CompilerParams(
            dimension_semantics=("arbitrary",)),
    )(gates_x, slen, w_hh.astype(jnp.bfloat16), w_out.astype(jnp.float32),
      b_out.astype(jnp.float32))

    return probs, indices, (h_last, c_last)


# speedup vs baseline: 1.3553x; 1.3553x over previous
import jax
import jax.numpy as jnp
from jax.experimental import pallas as pl
from jax.experimental.pallas import tpu as pltpu


_TOPK = 10
_DIST = 8


def _policy_kernel(idx_ref, emb_hbm, wih_ref, b_ref, slen_ref, whh_ref,
                   wout_hbm, bout_ref,
                   probs_hbm, h_ref, c_ref, idx_out_ref,
                   se_vmem, wout_vmem, probs_vmem,
                   gsems, wout_sem, probs_sem):
    TB, E = se_vmem.shape
    Bb, H = h_ref.shape[1], h_ref.shape[2]
    C = wout_vmem.shape[1]
    T = TB // Bb

    def issue(t):
        for j in range(Bb):
            i = t * Bb + j
            r = idx_ref[i]
            pltpu.make_async_copy(emb_hbm.at[pl.ds(r, 1)],
                                  se_vmem.at[pl.ds(i, 1)],
                                  gsems.at[t]).start()

    for t0 in range(min(_DIST, T)):
        issue(t0)
    wout_cp = pltpu.make_async_copy(wout_hbm, wout_vmem, wout_sem)
    wout_cp.start()

    h = jnp.zeros((Bb, H), jnp.bfloat16)
    c = jnp.zeros((Bb, H), jnp.float32)
    slen = slen_ref[...]

    for t in range(T):
        if t + _DIST < T:
            issue(t + _DIST)
        pltpu.make_async_copy(
            se_vmem.at[pl.ds(t * Bb, Bb)],
            se_vmem.at[pl.ds(t * Bb, Bb)],
            gsems.at[t]).wait()
        gx_t = (jnp.dot(se_vmem[t * Bb:(t + 1) * Bb], wih_ref[...],
                        preferred_element_type=jnp.float32)
                + b_ref[...]).astype(jnp.bfloat16)
        gates = gx_t.astype(jnp.float32) + jnp.dot(
            h, whh_ref[...], preferred_element_type=jnp.float32)
        i_g = jax.nn.sigmoid(gates[:, 0 * H:1 * H])
        f_g = jax.nn.sigmoid(gates[:, 1 * H:2 * H])
        g_g = jnp.tanh(gates[:, 2 * H:3 * H])
        o_g = jax.nn.sigmoid(gates[:, 3 * H:4 * H])
        c_new = f_g * c + i_g * g_g
        h_new = (o_g * jnp.tanh(c_new)).astype(jnp.bfloat16)
        valid = t < slen
        h = jnp.where(valid, h_new, h)
        c = jnp.where(valid, c_new, c)

    wout_cp.wait()
    hf = h.astype(jnp.float32)
    logits = jnp.dot(hf, wout_vmem[...],
                     preferred_element_type=jnp.float32) + bout_ref[...]
    m = jnp.max(logits, axis=1, keepdims=True)
    e = jnp.exp(logits - m)
    probs_vmem[...] = e * (1.0 / jnp.sum(e, axis=1, keepdims=True))
    probs_cp = pltpu.make_async_copy(probs_vmem, probs_hbm, probs_sem)
    probs_cp.start()
    h_ref[0] = hf
    c_ref[0] = c

    lane = jax.lax.broadcasted_iota(jnp.int32, (Bb, C), 1)
    vals = logits
    mk = m
    for k in range(_TOPK):
        idx_k = jnp.min(jnp.where(vals == mk, lane, C), axis=1, keepdims=True)
        idx_out_ref[:, k:k + 1] = idx_k
        if k + 1 < _TOPK:
            vals = jnp.where(lane == idx_k, -jnp.inf, vals)
            mk = jnp.max(vals, axis=1, keepdims=True)

    probs_cp.wait()


def kernel(seq_idx, seq_len, embedding, w_ih, w_hh, b, w_out, b_out):
    T, B = seq_idx.shape
    E = embedding.shape[1]
    H = w_hh.shape[0]
    C = w_out.shape[1]

    flat_idx = seq_idx.reshape(T * B)
    slen = seq_len.astype(jnp.int32)[:, None]

    grid_spec = pltpu.PrefetchScalarGridSpec(
        num_scalar_prefetch=1,
        grid=(1,),
        in_specs=[
            pl.BlockSpec(memory_space=pl.ANY),
            pl.BlockSpec((E, 4 * H), lambda i, ir: (0, 0)),
            pl.BlockSpec((1, 4 * H), lambda i, ir: (0, 0)),
            pl.BlockSpec((B, 1), lambda i, ir: (i, 0)),
            pl.BlockSpec((H, 4 * H), lambda i, ir: (0, 0)),
            pl.BlockSpec(memory_space=pl.ANY),
            pl.BlockSpec((1, C), lambda i, ir: (0, 0)),
        ],
        out_specs=(
            pl.BlockSpec(memory_space=pl.ANY),
            pl.BlockSpec((1, B, H), lambda i, ir: (0, i, 0)),
            pl.BlockSpec((1, B, H), lambda i, ir: (0, i, 0)),
            pl.BlockSpec((B, _TOPK), lambda i, ir: (i, 0)),
        ),
        scratch_shapes=[
            pltpu.VMEM((T * B, E), jnp.float32),
            pltpu.VMEM((H, C), jnp.float32),
            pltpu.VMEM((B, C), jnp.float32),
            pltpu.SemaphoreType.DMA((T,)),
            pltpu.SemaphoreType.DMA,
            pltpu.SemaphoreType.DMA,
        ],
    )

    probs, h_last, c_last, indices = pl.pallas_call(
        _policy_kernel,
        out_shape=(
            jax.ShapeDtypeStruct((B, C), jnp.float32),
            jax.ShapeDtypeStruct((1, B, H), jnp.float32),
            jax.ShapeDtypeStruct((1, B, H), jnp.float32),
            jax.ShapeDtypeStruct((B, _TOPK), jnp.int32),
        ),
        grid_spec=grid_spec,
        compiler_params=pltpu.CompilerParams(
            dimension_semantics=("arbitrary",),
            disable_bounds_checks=True,
            vmem_limit_bytes=61_000_000,
        ),
    )(flat_idx, embedding, w_ih.astype(jnp.float32), b.astype(jnp.float32),
      slen, w_hh.astype(jnp.bfloat16), w_out.astype(jnp.float32),
      b_out.astype(jnp.float32))

    return probs, indices, (h_last, c_last)


# speedup vs baseline: 1.3780x; 1.0167x over previous
import jax
import jax.numpy as jnp
from jax.experimental import pallas as pl
from jax.experimental.pallas import tpu as pltpu


_TOPK = 10
_DIST = 8


def _policy_kernel(idx_ref, emb_hbm, wih_ref, b_ref, slen_ref, whh_ref,
                   wout_hbm, bout_ref,
                   probs_hbm, h_ref, c_ref, idx_out_ref,
                   se_vmem, wout_vmem, probs_vmem,
                   gsems, wout_sem, probs_sem):
    TB, E = se_vmem.shape
    Bb, H = h_ref.shape[1], h_ref.shape[2]
    C = wout_vmem.shape[1]
    T = TB // Bb

    def issue(t):
        for j in range(Bb):
            i = t * Bb + j
            r = idx_ref[i]
            pltpu.make_async_copy(emb_hbm.at[pl.ds(r, 1)],
                                  se_vmem.at[pl.ds(i, 1)],
                                  gsems.at[t]).start(priority=j % 2)

    for t0 in range(min(_DIST, T)):
        issue(t0)
    wout_cp = pltpu.make_async_copy(wout_hbm, wout_vmem, wout_sem)
    wout_cp.start()

    h = jnp.zeros((Bb, H), jnp.bfloat16)
    c = jnp.zeros((Bb, H), jnp.float32)
    slen = slen_ref[...]

    for t in range(T):
        if t + _DIST < T:
            issue(t + _DIST)
        pltpu.make_async_copy(
            se_vmem.at[pl.ds(t * Bb, Bb)],
            se_vmem.at[pl.ds(t * Bb, Bb)],
            gsems.at[t]).wait()
        gx_t = (jnp.dot(se_vmem[t * Bb:(t + 1) * Bb], wih_ref[...],
                        preferred_element_type=jnp.float32)
                + b_ref[...]).astype(jnp.bfloat16)
        gates = gx_t.astype(jnp.float32) + jnp.dot(
            h, whh_ref[...], preferred_element_type=jnp.float32)
        i_g = jax.nn.sigmoid(gates[:, 0 * H:1 * H])
        f_g = jax.nn.sigmoid(gates[:, 1 * H:2 * H])
        g_g = jnp.tanh(gates[:, 2 * H:3 * H])
        o_g = jax.nn.sigmoid(gates[:, 3 * H:4 * H])
        c_new = f_g * c + i_g * g_g
        h_new = (o_g * jnp.tanh(c_new)).astype(jnp.bfloat16)
        valid = t < slen
        h = jnp.where(valid, h_new, h)
        c = jnp.where(valid, c_new, c)

    wout_cp.wait()
    hf = h.astype(jnp.float32)
    logits = jnp.dot(hf, wout_vmem[...],
                     preferred_element_type=jnp.float32) + bout_ref[...]
    m = jnp.max(logits, axis=1, keepdims=True)
    e = jnp.exp(logits - m)
    probs_vmem[...] = e * (1.0 / jnp.sum(e, axis=1, keepdims=True))
    probs_cp = pltpu.make_async_copy(probs_vmem, probs_hbm, probs_sem)
    probs_cp.start()
    h_ref[0] = hf
    c_ref[0] = c

    lane = jax.lax.broadcasted_iota(jnp.int32, (Bb, C), 1)
    vals = logits
    mk = m
    for k in range(_TOPK):
        idx_k = jnp.min(jnp.where(vals == mk, lane, C), axis=1, keepdims=True)
        idx_out_ref[:, k:k + 1] = idx_k
        if k + 1 < _TOPK:
            vals = jnp.where(lane == idx_k, -jnp.inf, vals)
            mk = jnp.max(vals, axis=1, keepdims=True)

    probs_cp.wait()


def kernel(seq_idx, seq_len, embedding, w_ih, w_hh, b, w_out, b_out):
    T, B = seq_idx.shape
    E = embedding.shape[1]
    H = w_hh.shape[0]
    C = w_out.shape[1]

    flat_idx = seq_idx.reshape(T * B)
    slen = seq_len.astype(jnp.int32)[:, None]

    grid_spec = pltpu.PrefetchScalarGridSpec(
        num_scalar_prefetch=1,
        grid=(1,),
        in_specs=[
            pl.BlockSpec(memory_space=pl.ANY),
            pl.BlockSpec((E, 4 * H), lambda i, ir: (0, 0)),
            pl.BlockSpec((1, 4 * H), lambda i, ir: (0, 0)),
            pl.BlockSpec((B, 1), lambda i, ir: (i, 0)),
            pl.BlockSpec((H, 4 * H), lambda i, ir: (0, 0)),
            pl.BlockSpec(memory_space=pl.ANY),
            pl.BlockSpec((1, C), lambda i, ir: (0, 0)),
        ],
        out_specs=(
            pl.BlockSpec(memory_space=pl.ANY),
            pl.BlockSpec((1, B, H), lambda i, ir: (0, i, 0)),
            pl.BlockSpec((1, B, H), lambda i, ir: (0, i, 0)),
            pl.BlockSpec((B, _TOPK), lambda i, ir: (i, 0)),
        ),
        scratch_shapes=[
            pltpu.VMEM((T * B, E), jnp.float32),
            pltpu.VMEM((H, C), jnp.float32),
            pltpu.VMEM((B, C), jnp.float32),
            pltpu.SemaphoreType.DMA((T,)),
            pltpu.SemaphoreType.DMA,
            pltpu.SemaphoreType.DMA,
        ],
    )

    probs, h_last, c_last, indices = pl.pallas_call(
        _policy_kernel,
        out_shape=(
            jax.ShapeDtypeStruct((B, C), jnp.float32),
            jax.ShapeDtypeStruct((1, B, H), jnp.float32),
            jax.ShapeDtypeStruct((1, B, H), jnp.float32),
            jax.ShapeDtypeStruct((B, _TOPK), jnp.int32),
        ),
        grid_spec=grid_spec,
        compiler_params=pltpu.CompilerParams(
            dimension_semantics=("arbitrary",),
            disable_bounds_checks=True,
            vmem_limit_bytes=61_000_000,
        ),
    )(flat_idx, embedding, w_ih.astype(jnp.float32), b.astype(jnp.float32),
      slen, w_hh.astype(jnp.bfloat16), w_out.astype(jnp.float32),
      b_out.astype(jnp.float32))

    return probs, indices, (h_last, c_last)
